# baseline (speedup 1.0000x reference)
import jax
import jax.numpy as jnp
from jax.experimental import pallas as pl

CHUNK = 512
NSTREAMS = 2


def _dot_mask(x, wt):
    logits = jax.lax.dot_general(
        x,
        wt,
        dimension_numbers=(((1,), (0,)), ((), ())),
        preferred_element_type=jnp.float32,
        precision=jax.lax.Precision.DEFAULT,
    )
    return (logits > 0.0).astype(jnp.int32)


def _gate_block(xa_ref, xb_ref, wt_ref, o_ref):
    wt = wt_ref[...]
    o_ref[0:CHUNK, :] = _dot_mask(xa_ref[...], wt)
    o_ref[CHUNK : 2 * CHUNK, :] = _dot_mask(xb_ref[...], wt)


@jax.jit
def kernel(cls_hidden_states, gate_w):
    tokens, hidden = cls_hidden_states.shape
    num_experts = gate_w.shape[0]
    wt = gate_w.T

    grid = (tokens // (CHUNK * NSTREAMS),)
    return pl.pallas_call(
        _gate_block,
        grid=grid,
        in_specs=[
            pl.BlockSpec((CHUNK, hidden), lambda i: (2 * i, 0)),
            pl.BlockSpec((CHUNK, hidden), lambda i: (2 * i + 1, 0)),
            pl.BlockSpec((hidden, num_experts), lambda i: (0, 0)),
        ],
        out_specs=pl.BlockSpec((CHUNK * NSTREAMS, num_experts), lambda i: (i, 0)),
        out_shape=jax.ShapeDtypeStruct((tokens, num_experts), jnp.int32),
    )(cls_hidden_states, cls_hidden_states, wt)
